# baseline (speedup 1.0000x reference)
import functools

import numpy as np
import jax
import jax.numpy as jnp
from jax import lax
from jax.experimental import pallas as pl
from jax.experimental.pallas import tpu as pltpu
from jax.experimental.pallas import tpu_sc as plsc

_L = 16
_B = 64
_N = _L ** 3
_NCG = 8
_NPG = 4
_PP = _N // _NPG
_CHP = 16


def _np_cond_matrix(roll_first: bool) -> np.ndarray:
    m = np.zeros((_L, _L), np.int64)
    for s in range(_L):
        v = np.zeros(_L, np.int64)
        v[s] = 1
        if roll_first:
            v = np.roll(v, 1)
        w = np.roll(np.flip(v), 1)
        m[:, s] = np.roll(np.cumsum(w), 1)
    return m


def _np_perm_tables() -> np.ndarray:
    def swap(t):
        s = t.shape
        return t.transpose(3, 2, 1, 0).reshape(s)[[0, 6, 2, 4, 3, 5, 1, 7]]

    base = np.arange(64).reshape(8, 2, 2, 2)
    tbl = np.zeros((64, 64), np.int32)
    for code in range(64):
        v = base.copy()
        for a in range(3):
            if (code >> (2 * a)) & 1:
                v = swap(np.roll(swap(v), 1, axis=-3 + a))
            if (code >> (2 * a + 1)) & 1:
                v = np.roll(v, 1, axis=-1 - a)
        tbl[code] = v.reshape(64)
    return tbl


def _np_code_mats():
    mc = _np_cond_matrix(False)
    mcr = _np_cond_matrix(True)
    p = np.arange(_N)
    pi, pj, pk = p // 256, (p // 16) % 16, p % 16
    g = np.arange(256) // 16
    t = np.arange(256) % 16
    r16 = np.arange(16)
    reduce_mats = [
        ((pj[:, None] == g[None, :]) * mc[t[None, :], pk[:, None]]),
        ((pi[:, None] == g[None, :]) * mc[t[None, :], pk[:, None]]),
        ((pi[:, None] == g[None, :]) * mc[t[None, :], pj[:, None]]),
        mcr[r16[None, :], pk[:, None]],
        mc[r16[None, :], pi[:, None]],
        mc[r16[None, :], pj[:, None]],
    ]
    expand_mats = [
        ((g[:, None] == pk[None, :]) & (t[:, None] == pj[None, :])) * 1,
        ((g[:, None] == pi[None, :]) & (t[:, None] == pk[None, :])) * 4,
        ((g[:, None] == pj[None, :]) & (t[:, None] == pi[None, :])) * 16,
        (r16[:, None] == pk[None, :]) * 2,
        (r16[:, None] == pi[None, :]) * 8,
        (r16[:, None] == pj[None, :]) * 32,
    ]
    return ([np.asarray(m, np.float32) for m in reduce_mats],
            [np.asarray(m.T, np.float32) for m in expand_mats])


_REDUCE_MATS, _EXPANDT_MATS = _np_code_mats()
_TBL = _np_perm_tables()
_WC = ((_TBL[:, None, :] == np.arange(64)[None, :, None])
       .astype(np.float32).reshape(64 * 64, 64) / float(_N)).astype(np.float32)


def _codes_t_body(syn_ref, cz0, cz1, cz2, cx0, cx1, cx2,
                  ez0t, ez1t, ez2t, ex0t, ex1t, ex2t, out_ref):
    bf = jnp.bfloat16
    f32 = jnp.float32
    s = syn_ref[...].astype(bf)
    parts = [s[:, :_N], s[:, _N:2 * _N], s[:, 2 * _N:3 * _N], s[:, 3 * _N:]]

    def mm(a, b):
        return jax.lax.dot_general(a, b, (((1,), (0,)), ((), ())),
                                   preferred_element_type=f32)

    def bits_t(pre):
        return jnp.transpose((pre.astype(jnp.int32) & 1).astype(bf), (1, 0))

    code = (mm(ez0t[...], bits_t(mm(parts[0], cz0[...])))
            + mm(ez1t[...], bits_t(mm(parts[0], cz1[...])))
            + mm(ez2t[...], bits_t(mm(parts[0], cz2[...])))
            + mm(ex0t[...], bits_t(mm(parts[1], cx0[...])))
            + mm(ex1t[...], bits_t(mm(parts[2], cx1[...])))
            + mm(ex2t[...], bits_t(mm(parts[3], cx2[...]))))
    out_ref[...] = code.astype(jnp.int32)


def _sc_bucket_body(x_hbm, codet_hbm, out_hbm,
                    codes_v, rows0, rows1, tab_v, sem0, sem1):
    wid = lax.axis_index("s") * 2 + lax.axis_index("c")
    cg = wid // _NPG
    pg = lax.rem(wid, _NPG)
    p0 = pg * _PP

    def zbody(i, c):
        tab_v[pl.ds(i * 16, 16)] = jnp.zeros((16,), jnp.float32)
        return c

    lax.fori_loop(0, (64 * 64 * 8) // 16, zbody, 0)

    iota = lax.broadcasted_iota(jnp.int32, (16,), 0)

    rows = [rows0, rows1]
    sems = [sem0, sem1]
    nch = _PP // _CHP

    def start_copy(ch, buf, sem):
        return pltpu.async_copy(
            x_hbm.at[pl.ds(p0 + ch * _CHP, _CHP), pl.ds(cg * 8, 8), :],
            buf, sem)

    cp = start_copy(0, rows0, sem0)
    for ch in range(nch):
        pltpu.sync_copy(codet_hbm.at[pl.ds(p0 + ch * _CHP, _CHP)], codes_v)
        cp.wait()
        if ch + 1 < nch:
            cp = start_copy(ch + 1, rows[(ch + 1) % 2], sems[(ch + 1) % 2])
        buf = rows[ch % 2]

        def pbody(p, c):
            for lg in range(4):
                cvec = codes_v[p, pl.ds(lg * 16, 16)]
                base = (lg * 16 + iota) * 512 + cvec * 8
                for cl in range(8):
                    v = buf[p, cl, pl.ds(lg * 16, 16)]
                    plsc.addupdate_scatter(tab_v, [base + cl], v)
            return c

        lax.fori_loop(0, _CHP, pbody, 0)

    pltpu.sync_copy(tab_v, out_hbm.at[pg, cg])


_SC_BUCKET_CACHE = []


def _sc_bucket():
    if not _SC_BUCKET_CACHE:
        _SC_BUCKET_CACHE.append(functools.partial(
            pl.kernel,
            out_type=jax.ShapeDtypeStruct((_NPG, _NCG, 64 * 64 * 8),
                                          jnp.float32),
            mesh=plsc.VectorSubcoreMesh(core_axis_name="c",
                                        subcore_axis_name="s",
                                        num_cores=2, num_subcores=16),
            compiler_params=pltpu.CompilerParams(needs_layout_passes=False),
            scratch_types=[
                pltpu.VMEM((_CHP, 64), jnp.int32),
                pltpu.VMEM((_CHP, 8, 64), jnp.float32),
                pltpu.VMEM((_CHP, 8, 64), jnp.float32),
                pltpu.VMEM((64 * 64 * 8,), jnp.float32),
                pltpu.SemaphoreType.DMA,
                pltpu.SemaphoreType.DMA,
            ],
        )(_sc_bucket_body))
    return _SC_BUCKET_CACHE[0]


def _combine_body(s_ref, w_ref, o_ref):
    o_ref[...] = jax.lax.dot_general(
        s_ref[...], w_ref[...], (((1,), (0,)), ((), ())),
        preferred_element_type=jnp.float32)


def _sum_partials_body(p_ref, o_ref):
    acc = jnp.zeros((64, 4096), jnp.float32)
    for pg in range(_NPG):
        for cg in range(_NCG):
            part = p_ref[pg, cg]
            acc = acc + part
    o_ref[...] = acc


def kernel(x, syndrome):
    b, n = _B, _N
    bf = jnp.bfloat16
    consts = ([jnp.asarray(m, bf) for m in _REDUCE_MATS]
              + [jnp.asarray(m, bf) for m in _EXPANDT_MATS])
    codet = pl.pallas_call(
        _codes_t_body,
        out_shape=jax.ShapeDtypeStruct((n, b), jnp.int32),
    )(syndrome, *consts)

    xv = x.transpose(1, 2, 3, 4, 5, 6, 7, 0).reshape(n, 64, b)
    parts = _sc_bucket()(xv, codet)
    sp = parts.reshape(_NPG, _NCG, 64, 64, 8).sum(axis=0)
    s = sp.transpose(1, 2, 0, 3).reshape(b, 64 * 64)

    out = pl.pallas_call(
        _combine_body,
        out_shape=jax.ShapeDtypeStruct((b, 64), jnp.float32),
    )(s, jnp.asarray(_WC))
    return out.reshape(b, 8, 2, 2, 2)


# speedup vs baseline: 4.4486x; 4.4486x over previous
import numpy as np
import jax
import jax.numpy as jnp
from jax.experimental import pallas as pl

_L = 16
_B = 64
_N = _L ** 3
_P = 256


def _np_cond_matrix(roll_first: bool) -> np.ndarray:
    m = np.zeros((_L, _L), np.int64)
    for s in range(_L):
        v = np.zeros(_L, np.int64)
        v[s] = 1
        if roll_first:
            v = np.roll(v, 1)
        w = np.roll(np.flip(v), 1)
        m[:, s] = np.roll(np.cumsum(w), 1)
    return m


def _np_perm_tables() -> np.ndarray:
    def swap(t):
        s = t.shape
        return t.transpose(3, 2, 1, 0).reshape(s)[[0, 6, 2, 4, 3, 5, 1, 7]]

    base = np.arange(64).reshape(8, 2, 2, 2)
    tbl = np.zeros((64, 64), np.int32)
    for code in range(64):
        v = base.copy()
        for a in range(3):
            if (code >> (2 * a)) & 1:
                v = swap(np.roll(swap(v), 1, axis=-3 + a))
            if (code >> (2 * a + 1)) & 1:
                v = np.roll(v, 1, axis=-1 - a)
        tbl[code] = v.reshape(64)
    return tbl


def _np_code_mats():
    mc = _np_cond_matrix(False)
    mcr = _np_cond_matrix(True)
    p = np.arange(_N)
    pi, pj, pk = p // 256, (p // 16) % 16, p % 16
    g = np.arange(256) // 16
    t = np.arange(256) % 16
    r16 = np.arange(16)
    reduce_mats = [
        ((pj[:, None] == g[None, :]) * mc[t[None, :], pk[:, None]]),
        ((pi[:, None] == g[None, :]) * mc[t[None, :], pk[:, None]]),
        ((pi[:, None] == g[None, :]) * mc[t[None, :], pj[:, None]]),
        mcr[r16[None, :], pk[:, None]],
        mc[r16[None, :], pi[:, None]],
        mc[r16[None, :], pj[:, None]],
    ]
    expand_mats = [
        ((g[:, None] == pk[None, :]) & (t[:, None] == pj[None, :])) * 1,
        ((g[:, None] == pi[None, :]) & (t[:, None] == pk[None, :])) * 4,
        ((g[:, None] == pj[None, :]) & (t[:, None] == pi[None, :])) * 16,
        (r16[:, None] == pk[None, :]) * 2,
        (r16[:, None] == pi[None, :]) * 8,
        (r16[:, None] == pj[None, :]) * 32,
    ]
    return ([np.asarray(m, np.float32) for m in reduce_mats],
            [np.asarray(m.T, np.float32) for m in expand_mats])


_REDUCE_MATS, _EXPANDT_MATS = _np_code_mats()
_TBL = _np_perm_tables()
_WC = ((_TBL[:, None, :] == np.arange(64)[None, :, None])
       .astype(np.float32).reshape(64 * 64, 64) / float(_N)).astype(np.float32)


def _codes_t_body(syn_ref, cz0, cz1, cz2, cx0, cx1, cx2,
                  ez0t, ez1t, ez2t, ex0t, ex1t, ex2t, out_ref):
    bf = jnp.bfloat16
    f32 = jnp.float32
    s = syn_ref[...].astype(bf)
    parts = [s[:, :_N], s[:, _N:2 * _N], s[:, 2 * _N:3 * _N], s[:, 3 * _N:]]

    def mm(a, b):
        return jax.lax.dot_general(a, b, (((1,), (0,)), ((), ())),
                                   preferred_element_type=f32)

    def bits_t(pre):
        return jnp.transpose((pre.astype(jnp.int32) & 1).astype(bf), (1, 0))

    code = (mm(ez0t[...], bits_t(mm(parts[0], cz0[...])))
            + mm(ez1t[...], bits_t(mm(parts[0], cz1[...])))
            + mm(ez2t[...], bits_t(mm(parts[0], cz2[...])))
            + mm(ex0t[...], bits_t(mm(parts[1], cx0[...])))
            + mm(ex1t[...], bits_t(mm(parts[2], cx1[...])))
            + mm(ex2t[...], bits_t(mm(parts[3], cx2[...]))))
    out_ref[...] = code.astype(jnp.int32)


def _bucket_body(ct_ref, x_ref, o_ref):
    i = pl.program_id(0)
    ctt = jnp.transpose(ct_ref[...], (1, 0))
    arrt = jnp.transpose(x_ref[...], (2, 0, 1))
    giota = jax.lax.broadcasted_iota(jnp.int32, (64, _P), 0)

    @pl.when(i == 0)
    def _():
        o_ref[...] = jnp.zeros_like(o_ref)

    for b in range(_B):
        cb = ctt[b:b + 1, :]
        at = (jnp.broadcast_to(cb, (64, _P)) == giota).astype(jnp.bfloat16)
        xb = arrt[b:b + 1].reshape(_P, 64).astype(jnp.bfloat16)
        sb = jax.lax.dot_general(at, xb, (((1,), (0,)), ((), ())),
                                 preferred_element_type=jnp.float32)
        o_ref[b, :, :] = o_ref[b, :, :] + sb


def _combine_body(s_ref, w_ref, o_ref):
    o_ref[...] = jax.lax.dot_general(
        s_ref[...], w_ref[...], (((1,), (0,)), ((), ())),
        preferred_element_type=jnp.float32)


def kernel(x, syndrome):
    b, n = _B, _N
    bf = jnp.bfloat16
    consts = ([jnp.asarray(m, bf) for m in _REDUCE_MATS]
              + [jnp.asarray(m, bf) for m in _EXPANDT_MATS])
    codet = pl.pallas_call(
        _codes_t_body,
        out_shape=jax.ShapeDtypeStruct((n, b), jnp.int32),
    )(syndrome, *consts)

    xv = x.transpose(1, 2, 3, 4, 5, 6, 7, 0).reshape(n, 64, b)
    s = pl.pallas_call(
        _bucket_body,
        grid=(n // _P,),
        in_specs=[
            pl.BlockSpec((_P, 64), lambda i: (i, 0)),
            pl.BlockSpec((_P, 64, 64), lambda i: (i, 0, 0)),
        ],
        out_specs=pl.BlockSpec((64, 64, 64), lambda i: (0, 0, 0)),
        out_shape=jax.ShapeDtypeStruct((64, 64, 64), jnp.float32),
    )(codet, xv)

    out = pl.pallas_call(
        _combine_body,
        out_shape=jax.ShapeDtypeStruct((b, 64), jnp.float32),
    )(s.reshape(b, 64 * 64), jnp.asarray(_WC))
    return out.reshape(b, 8, 2, 2, 2)
